# baseline (speedup 1.0000x reference)
import jax
import jax.numpy as jnp
import numpy as np
from jax.experimental import pallas as pl
from jax.experimental.pallas import tpu as pltpu

B = 2
C = 128
HW = 40000
NU = HW + 1
HEADS = 8
HDIM = C // HEADS
EMBED = 512
SCALE = 1.0 / np.sqrt(HDIM)

BLK_A = 4096
BLK_B = 2560
BLK_C = 10240
PC_LAST = HW - (pl.cdiv(HW, BLK_B) - 1) * BLK_B
POS_LAST = NU - (pl.cdiv(NU, BLK_B) - 1) * BLK_B


def _mean_kernel(pc_ref, out_ref):
    i = pl.program_id(0)
    rem = HW - i * BLK_A
    lane = jax.lax.broadcasted_iota(jnp.int32, (B, C, BLK_A), 2)
    s = jnp.sum(jnp.where(lane < rem, pc_ref[...], 0.0), axis=2)

    @pl.when(i == 0)
    def _():
        out_ref[...] = s

    @pl.when(i > 0)
    def _():
        out_ref[...] += s


def _flash_kernel(pc_ref, pos_ref, mean_ref, meant_ref, pos0_ref, qw_ref,
                  qb_ref, kw_ref, kb_ref,
                  lt_ref, m_out, d_out, acc_out,
                  wf_s, kb_s, m_s, d_s, acc_s, lpcc_s, pcc_s):
    i = pl.program_id(0)
    nsteps = pl.num_programs(0)
    rem_u = NU - i * BLK_B
    lane_hn = jax.lax.broadcasted_iota(jnp.int32, (HEADS, BLK_B), 1)

    @pl.when(i == nsteps - 1)
    def _():
        pc_ref[:, :, pl.ds(PC_LAST, BLK_B - PC_LAST)] = jnp.zeros(
            (B, C, BLK_B - PC_LAST), jnp.float32)
        pos_ref[pl.ds(POS_LAST, BLK_B - POS_LAST), :] = jnp.zeros(
            (BLK_B - POS_LAST, C), jnp.float32)

    @pl.when(i == 0)
    def _():
        x0q = mean_ref[...] * (1.0 / HW) + pos0_ref[...]
        q = jax.lax.dot_general(x0q, qw_ref[...], (((1,), (1,)), ((), ())))
        q = (q + qb_ref[...]) * SCALE
        h_i = jax.lax.broadcasted_iota(jnp.int32, (HEADS, C), 0)
        j_i = jax.lax.broadcasted_iota(jnp.int32, (HEADS, C), 1)
        head_sel = (h_i == j_i // HDIM).astype(jnp.float32)
        for b in range(B):
            hq = head_sel * q[b:b + 1]
            wf = jax.lax.dot_general(hq, kw_ref[...],
                                     (((1,), (0,)), ((), ())))
            wf_s[b] = wf
            kb_s[b] = jax.lax.dot_general(hq, kb_ref[...],
                                          (((1,), (1,)), ((), ())))
            m_s[b] = jnp.full((HEADS, 1), -1e30, jnp.float32)
            d_s[b] = jnp.zeros((HEADS, 1), jnp.float32)
            acc_s[b] = jnp.zeros((HEADS, C), jnp.float32)
            mcol = meant_ref[:, b:b + 1] * (1.0 / HW)
            pcc_s[b] = mcol
            lpcc_s[b] = jax.lax.dot_general(wf, mcol,
                                            (((1,), (0,)), ((), ())))

    pos = pos_ref[...]
    for b in range(B):
        pc = pc_ref[b]
        wf = wf_s[b]
        lpc = jax.lax.dot_general(wf, pc, (((1,), (0,)), ((), ())))
        lpe = jax.lax.dot_general(wf, pos, (((1,), (1,)), ((), ())))
        lpc_u = jnp.concatenate([lpcc_s[b], lpc[:, :BLK_B - 1]], axis=1)
        logits = lpc_u + lpe + kb_s[b]
        logits = jnp.where(lane_hn < rem_u, logits, -1e30)
        lt_ref[b] = logits

        m_new = jnp.maximum(m_s[b], jnp.max(logits, axis=1, keepdims=True))
        alpha = jnp.exp(m_s[b] - m_new)
        e = jnp.exp(logits - m_new)
        d_new = d_s[b] * alpha + jnp.sum(e, axis=1, keepdims=True)
        a_pe = jax.lax.dot_general(e, pos, (((1,), (0,)), ((), ())))
        a_pc = jax.lax.dot_general(e[:, 1:], pc[:, :BLK_B - 1],
                                   (((1,), (1,)), ((), ())))
        a_bd = jax.lax.dot_general(e[:, 0:1], pcc_s[b],
                                   (((1,), (1,)), ((), ())))
        acc_new = acc_s[b] * alpha + a_pe + a_pc + a_bd
        m_s[b] = m_new
        d_s[b] = d_new
        acc_s[b] = acc_new
        lpcc_s[b] = lpc[:, BLK_B - 1:]
        pcc_s[b] = pc[:, BLK_B - 1:]

        @pl.when(i == nsteps - 1)
        def _():
            m_out[b] = m_new
            d_out[b] = d_new
            acc_out[b] = acc_new


def _final_kernel(lt_ref, m_ref, d_ref, acc_ref, vw_ref, vb_ref,
                  cw_ref, cb_ref, attn_ref, pooled_ref):
    i = pl.program_id(0)
    for b in range(B):
        rinv = 1.0 / d_ref[b]
        attn_ref[b] = jnp.exp(lt_ref[b] - m_ref[b]) * rinv

    @pl.when(i == 0)
    def _():
        j_i = jax.lax.broadcasted_iota(jnp.int32, (C, HEADS), 0)
        h_i = jax.lax.broadcasted_iota(jnp.int32, (C, HEADS), 1)
        head_sel = (j_i // HDIM == h_i).astype(jnp.float32)
        for b in range(B):
            s_x = acc_ref[b] * (1.0 / d_ref[b])
            s_sel = jax.lax.dot_general(head_sel, s_x,
                                        (((1,), (0,)), ((), ())))
            outv = jnp.sum(s_sel * vw_ref[...], axis=1, keepdims=True)
            outv = outv + vb_ref[...]
            pooled = jax.lax.dot_general(outv, cw_ref[...],
                                         (((0,), (1,)), ((), ())))
            pooled_ref[b:b + 1] = pooled + cb_ref[...]


def kernel(point_cloud, pos_emb, q_w, q_b, k_w, k_b, v_w, v_b, c_w, c_b):
    pc3 = point_cloud.reshape(B, C, HW)
    pos0 = pos_emb[0:1]
    qb2 = q_b.reshape(1, C)
    kb2 = k_b.reshape(1, C)
    vb2 = v_b.reshape(C, 1)
    cb2 = c_b.reshape(1, EMBED)
    f32 = jnp.float32

    mean_sum = pl.pallas_call(
        _mean_kernel,
        grid=(pl.cdiv(HW, BLK_A),),
        in_specs=[pl.BlockSpec((B, C, BLK_A), lambda i: (0, 0, i))],
        out_specs=pl.BlockSpec((B, C), lambda i: (0, 0)),
        out_shape=jax.ShapeDtypeStruct((B, C), f32),
    )(pc3)
    mean_t = mean_sum.T

    small = pl.BlockSpec((B, HEADS, 1), lambda i: (0, 0, 0))
    vmem = pltpu.VMEM
    lt, m, d, acc = pl.pallas_call(
        _flash_kernel,
        grid=(pl.cdiv(NU, BLK_B),),
        in_specs=[
            pl.BlockSpec((B, C, BLK_B), lambda i: (0, 0, i)),
            pl.BlockSpec((BLK_B, C), lambda i: (i, 0)),
            pl.BlockSpec((B, C), lambda i: (0, 0)),
            pl.BlockSpec((C, B), lambda i: (0, 0)),
            pl.BlockSpec((1, C), lambda i: (0, 0)),
            pl.BlockSpec((C, C), lambda i: (0, 0)),
            pl.BlockSpec((1, C), lambda i: (0, 0)),
            pl.BlockSpec((C, C), lambda i: (0, 0)),
            pl.BlockSpec((1, C), lambda i: (0, 0)),
        ],
        out_specs=[
            pl.BlockSpec((B, HEADS, BLK_B), lambda i: (0, 0, i)),
            small, small,
            pl.BlockSpec((B, HEADS, C), lambda i: (0, 0, 0)),
        ],
        out_shape=[
            jax.ShapeDtypeStruct((B, HEADS, NU), f32),
            jax.ShapeDtypeStruct((B, HEADS, 1), f32),
            jax.ShapeDtypeStruct((B, HEADS, 1), f32),
            jax.ShapeDtypeStruct((B, HEADS, C), f32),
        ],
        scratch_shapes=[
            vmem((B, HEADS, C), f32),
            vmem((B, HEADS, 1), f32),
            vmem((B, HEADS, 1), f32),
            vmem((B, HEADS, 1), f32),
            vmem((B, HEADS, C), f32),
            vmem((B, HEADS, 1), f32),
            vmem((B, C, 1), f32),
        ],
    )(pc3, pos_emb, mean_sum, mean_t, pos0, q_w, qb2, k_w, kb2)

    attn, pooled = pl.pallas_call(
        _final_kernel,
        grid=(pl.cdiv(NU, BLK_C),),
        in_specs=[
            pl.BlockSpec((B, HEADS, BLK_C), lambda i: (0, 0, i)),
            small, small,
            pl.BlockSpec((B, HEADS, C), lambda i: (0, 0, 0)),
            pl.BlockSpec((C, C), lambda i: (0, 0)),
            pl.BlockSpec((C, 1), lambda i: (0, 0)),
            pl.BlockSpec((EMBED, C), lambda i: (0, 0)),
            pl.BlockSpec((1, EMBED), lambda i: (0, 0)),
        ],
        out_specs=[
            pl.BlockSpec((B, HEADS, BLK_C), lambda i: (0, 0, i)),
            pl.BlockSpec((B, EMBED), lambda i: (0, 0)),
        ],
        out_shape=[
            jax.ShapeDtypeStruct((B, HEADS, NU), f32),
            jax.ShapeDtypeStruct((B, EMBED), f32),
        ],
    )(lt, m, d, acc, v_w, vb2, c_w, cb2)

    return pooled, attn


# speedup vs baseline: 1.2652x; 1.2652x over previous
import jax
import jax.numpy as jnp
import numpy as np
from jax.experimental import pallas as pl
from jax.experimental.pallas import tpu as pltpu

B = 2
C = 128
HW = 40000
NU = HW + 1
HEADS = 8
HDIM = C // HEADS
EMBED = 512
SCALE = 1.0 / np.sqrt(HDIM)

BLK_A = 8192
BLK_B = 10240
BLK_C = 20480
PC_LAST = HW - (pl.cdiv(HW, BLK_B) - 1) * BLK_B
POS_LAST = NU - (pl.cdiv(NU, BLK_B) - 1) * BLK_B
MEAN_LAST = HW - (pl.cdiv(HW, BLK_A) - 1) * BLK_A


def _mean_kernel(pc_ref, out_ref):
    i = pl.program_id(0)

    @pl.when(i == pl.num_programs(0) - 1)
    def _():
        pc_ref[:, :, pl.ds(MEAN_LAST, BLK_A - MEAN_LAST)] = jnp.zeros(
            (B, C, BLK_A - MEAN_LAST), jnp.float32)

    s = jnp.sum(pc_ref[...], axis=2, dtype=jnp.float32)

    @pl.when(i == 0)
    def _():
        out_ref[...] = s

    @pl.when(i > 0)
    def _():
        out_ref[...] += s


def _flash_kernel(pc_ref, pos_ref, mean_ref, meant_ref, pos0_ref, qw_ref,
                  qb_ref, kw_ref, kb_ref,
                  lt_ref, m_out, d_out, acc_out,
                  wf2_s, wfb_s, kb_s, m_s, d_s, acc_s, lpcc_s, pcc_s):
    i = pl.program_id(0)
    nsteps = pl.num_programs(0)
    rem_u = NU - i * BLK_B
    BH = B * HEADS
    BC = B * C
    lane_hn = jax.lax.broadcasted_iota(jnp.int32, (BH, BLK_B), 1)

    @pl.when(i == nsteps - 1)
    def _():
        pc_ref[:, :, pl.ds(PC_LAST, BLK_B - PC_LAST)] = jnp.zeros(
            (B, C, BLK_B - PC_LAST), jnp.float32)
        pos_ref[pl.ds(POS_LAST, BLK_B - POS_LAST), :] = jnp.zeros(
            (BLK_B - POS_LAST, C), jnp.float32)

    @pl.when(i == 0)
    def _():
        x0q = mean_ref[...] * (1.0 / HW) + pos0_ref[...]
        q = jax.lax.dot_general(x0q, qw_ref[...], (((1,), (1,)), ((), ())))
        q = (q + qb_ref[...]) * SCALE
        h_i = jax.lax.broadcasted_iota(jnp.int32, (HEADS, C), 0)
        j_i = jax.lax.broadcasted_iota(jnp.int32, (HEADS, C), 1)
        head_sel = (h_i == j_i // HDIM).astype(jnp.float32)
        wfs = []
        for b in range(B):
            hq = head_sel * q[b:b + 1]
            wfs.append(jax.lax.dot_general(hq, kw_ref[...],
                                           (((1,), (0,)), ((), ()))))
        wfb_s[...] = jnp.concatenate(wfs, axis=0)
        z8 = jnp.zeros((HEADS, C), jnp.float32)
        wf2_s[...] = jnp.concatenate(
            [jnp.concatenate([wfs[0], z8], axis=1),
             jnp.concatenate([z8, wfs[1]], axis=1)], axis=0)
        hq2 = jnp.concatenate([head_sel * q[0:1], head_sel * q[1:2]], axis=0)
        kb_s[...] = jax.lax.dot_general(hq2, kb_ref[...],
                                        (((1,), (1,)), ((), ())))
        m_s[...] = jnp.full((BH, 1), -1e30, jnp.float32)
        d_s[...] = jnp.zeros((BH, 1), jnp.float32)
        acc_s[...] = jnp.zeros((BH, C), jnp.float32)
        mcol2 = jnp.concatenate([meant_ref[:, 0:1], meant_ref[:, 1:2]],
                                axis=0) * (1.0 / HW)
        pcc_s[...] = mcol2
        lpcc_s[...] = jax.lax.dot_general(wf2_s[...], mcol2,
                                          (((1,), (0,)), ((), ())))

    pos = pos_ref[...]
    pc2 = pc_ref[...].reshape(BC, BLK_B)
    wf2 = wf2_s[...]
    lpc = jax.lax.dot_general(wf2, pc2, (((1,), (0,)), ((), ())))
    lpe = jax.lax.dot_general(wfb_s[...], pos, (((1,), (1,)), ((), ())))
    lpc_u = jnp.concatenate([lpcc_s[...], lpc[:, :BLK_B - 1]], axis=1)
    logits = lpc_u + lpe + kb_s[...]
    logits = jnp.where(lane_hn < rem_u, logits, -1e30)
    lt_ref[...] = logits.reshape(B, HEADS, BLK_B)

    m_new = jnp.maximum(m_s[...], jnp.max(logits, axis=1, keepdims=True))
    alpha = jnp.exp(m_s[...] - m_new)
    e = jnp.exp(logits - m_new)
    d_new = d_s[...] * alpha + jnp.sum(e, axis=1, keepdims=True)
    a_pe = jax.lax.dot_general(e, pos, (((1,), (0,)), ((), ())))
    a_pc2 = jax.lax.dot_general(e[:, 1:], pc2[:, :BLK_B - 1],
                                (((1,), (1,)), ((), ())))
    a_pc2 = a_pc2 + jax.lax.dot_general(e[:, 0:1], pcc_s[...],
                                        (((1,), (1,)), ((), ())))
    a_pc = jnp.concatenate([a_pc2[:HEADS, :C], a_pc2[HEADS:, C:]], axis=0)
    acc_new = acc_s[...] * alpha + a_pe + a_pc
    m_s[...] = m_new
    d_s[...] = d_new
    acc_s[...] = acc_new
    lpcc_s[...] = lpc[:, BLK_B - 1:]
    pcc_s[...] = pc2[:, BLK_B - 1:]

    @pl.when(i == nsteps - 1)
    def _():
        m_out[...] = m_new.reshape(B, HEADS, 1)
        d_out[...] = d_new.reshape(B, HEADS, 1)
        acc_out[...] = acc_new.reshape(B, HEADS, C)


def _final_kernel(lt_ref, m_ref, d_ref, acc_ref, vw_ref, vb_ref,
                  cw_ref, cb_ref, attn_ref, pooled_ref):
    i = pl.program_id(0)
    attn_ref[...] = jnp.exp(lt_ref[...] - m_ref[...]) * (1.0 / d_ref[...])

    @pl.when(i == 0)
    def _():
        j_i = jax.lax.broadcasted_iota(jnp.int32, (C, HEADS), 0)
        h_i = jax.lax.broadcasted_iota(jnp.int32, (C, HEADS), 1)
        head_sel = (j_i // HDIM == h_i).astype(jnp.float32)
        for b in range(B):
            s_x = acc_ref[b] * (1.0 / d_ref[b])
            s_sel = jax.lax.dot_general(head_sel, s_x,
                                        (((1,), (0,)), ((), ())))
            outv = jnp.sum(s_sel * vw_ref[...], axis=1, keepdims=True)
            outv = outv + vb_ref[...]
            pooled = jax.lax.dot_general(outv, cw_ref[...],
                                         (((0,), (1,)), ((), ())))
            pooled_ref[b:b + 1] = pooled + cb_ref[...]


def kernel(point_cloud, pos_emb, q_w, q_b, k_w, k_b, v_w, v_b, c_w, c_b):
    pc3 = point_cloud.reshape(B, C, HW)
    pos0 = pos_emb[0:1]
    qb2 = q_b.reshape(1, C)
    kb2 = k_b.reshape(1, C)
    vb2 = v_b.reshape(C, 1)
    cb2 = c_b.reshape(1, EMBED)
    f32 = jnp.float32

    mean_sum = pl.pallas_call(
        _mean_kernel,
        grid=(pl.cdiv(HW, BLK_A),),
        in_specs=[pl.BlockSpec((B, C, BLK_A), lambda i: (0, 0, i))],
        out_specs=pl.BlockSpec((B, C), lambda i: (0, 0)),
        out_shape=jax.ShapeDtypeStruct((B, C), f32),
    )(pc3)
    mean_t = mean_sum.T

    small = pl.BlockSpec((B, HEADS, 1), lambda i: (0, 0, 0))
    vmem = pltpu.VMEM
    lt, m, d, acc = pl.pallas_call(
        _flash_kernel,
        grid=(pl.cdiv(NU, BLK_B),),
        in_specs=[
            pl.BlockSpec((B, C, BLK_B), lambda i: (0, 0, i)),
            pl.BlockSpec((BLK_B, C), lambda i: (i, 0)),
            pl.BlockSpec((B, C), lambda i: (0, 0)),
            pl.BlockSpec((C, B), lambda i: (0, 0)),
            pl.BlockSpec((1, C), lambda i: (0, 0)),
            pl.BlockSpec((C, C), lambda i: (0, 0)),
            pl.BlockSpec((1, C), lambda i: (0, 0)),
            pl.BlockSpec((C, C), lambda i: (0, 0)),
            pl.BlockSpec((1, C), lambda i: (0, 0)),
        ],
        out_specs=[
            pl.BlockSpec((B, HEADS, BLK_B), lambda i: (0, 0, i)),
            small, small,
            pl.BlockSpec((B, HEADS, C), lambda i: (0, 0, 0)),
        ],
        out_shape=[
            jax.ShapeDtypeStruct((B, HEADS, NU), f32),
            jax.ShapeDtypeStruct((B, HEADS, 1), f32),
            jax.ShapeDtypeStruct((B, HEADS, 1), f32),
            jax.ShapeDtypeStruct((B, HEADS, C), f32),
        ],
        scratch_shapes=[
            vmem((B * HEADS, B * C), f32),
            vmem((B * HEADS, C), f32),
            vmem((B * HEADS, 1), f32),
            vmem((B * HEADS, 1), f32),
            vmem((B * HEADS, 1), f32),
            vmem((B * HEADS, C), f32),
            vmem((B * HEADS, 1), f32),
            vmem((B * C, 1), f32),
        ],
    )(pc3, pos_emb, mean_sum, mean_t, pos0, q_w, qb2, k_w, kb2)

    attn, pooled = pl.pallas_call(
        _final_kernel,
        grid=(pl.cdiv(NU, BLK_C),),
        in_specs=[
            pl.BlockSpec((B, HEADS, BLK_C), lambda i: (0, 0, i)),
            small, small,
            pl.BlockSpec((B, HEADS, C), lambda i: (0, 0, 0)),
            pl.BlockSpec((C, C), lambda i: (0, 0)),
            pl.BlockSpec((C, 1), lambda i: (0, 0)),
            pl.BlockSpec((EMBED, C), lambda i: (0, 0)),
            pl.BlockSpec((1, EMBED), lambda i: (0, 0)),
        ],
        out_specs=[
            pl.BlockSpec((B, HEADS, BLK_C), lambda i: (0, 0, i)),
            pl.BlockSpec((B, EMBED), lambda i: (0, 0)),
        ],
        out_shape=[
            jax.ShapeDtypeStruct((B, HEADS, NU), f32),
            jax.ShapeDtypeStruct((B, EMBED), f32),
        ],
    )(lt, m, d, acc, v_w, vb2, c_w, cb2)

    return pooled, attn
